# baseline (device time: 13172 ns/iter reference)
import jax
import jax.numpy as jnp
from jax import lax
from jax.experimental import pallas as pl
from jax.experimental.pallas import tpu as pltpu

N_DEV = 32


def kernel(x):
    m_per, n = x.shape

    def body(x_ref, out_ref, own_ref, comm_ref, send_sems, recv_sems):
        my = lax.axis_index("i")
        barrier_sem = pltpu.get_barrier_semaphore()

        for d in range(1, N_DEV):

            @pl.when(my - d >= 0)
            def _(d=d):
                pl.semaphore_signal(
                    barrier_sem,
                    inc=1,
                    device_id=(my - d,),
                    device_id_type=pl.DeviceIdType.MESH,
                )

        own_ref[0, :] = jnp.sum(x_ref[...], axis=0)

        for d in range(1, N_DEV):

            @pl.when(my + d < N_DEV)
            def _(d=d):
                pl.semaphore_wait(barrier_sem, 1)

        for d in range(1, N_DEV):

            @pl.when(my + d < N_DEV)
            def _(d=d):
                rdma = pltpu.make_async_remote_copy(
                    src_ref=own_ref,
                    dst_ref=comm_ref.at[my],
                    send_sem=send_sems.at[d - 1],
                    recv_sem=recv_sems.at[my],
                    device_id=(my + d,),
                    device_id_type=pl.DeviceIdType.MESH,
                )
                rdma.start()

        chunk = 128
        n_chunks = m_per // chunk
        row = lax.broadcasted_iota(jnp.int32, (chunk, chunk), 0)
        col = lax.broadcasted_iota(jnp.int32, (chunk, chunk), 1)
        tri = (row >= col).astype(jnp.bfloat16)
        running = jnp.zeros((1, n), jnp.float32)
        for c in range(n_chunks):
            seg = x_ref[c * chunk : (c + 1) * chunk, :].astype(jnp.bfloat16)
            scan_c = jax.lax.dot_general(
                tri, seg, (((1,), (0,)), ((), ())),
                preferred_element_type=jnp.float32,
            )
            out_ref[c * chunk : (c + 1) * chunk, :] = scan_c + running
            running = running + scan_c[-1:, :]

        for j in range(N_DEV - 1):

            @pl.when(j < my)
            def _(j=j):
                recv = pltpu.make_async_remote_copy(
                    src_ref=own_ref,
                    dst_ref=comm_ref.at[j],
                    send_sem=send_sems.at[0],
                    recv_sem=recv_sems.at[j],
                    device_id=(0,),
                    device_id_type=pl.DeviceIdType.MESH,
                )
                recv.wait_recv()

        slot_row = lax.broadcasted_iota(jnp.int32, (N_DEV, n), 0)
        totals = jnp.where(slot_row < my, comm_ref[:, 0, :], 0.0)
        prefix = jnp.sum(totals, axis=0, keepdims=True)

        out_ref[...] = out_ref[...] + prefix

        for d in range(1, N_DEV):

            @pl.when(my + d < N_DEV)
            def _(d=d):
                send = pltpu.make_async_remote_copy(
                    src_ref=own_ref,
                    dst_ref=comm_ref.at[my],
                    send_sem=send_sems.at[d - 1],
                    recv_sem=recv_sems.at[0],
                    device_id=(my + d,),
                    device_id_type=pl.DeviceIdType.MESH,
                )
                send.wait_send()

    return pl.pallas_call(
        body,
        out_shape=jax.ShapeDtypeStruct((m_per, n), jnp.float32),
        in_specs=[pl.BlockSpec(memory_space=pltpu.VMEM)],
        out_specs=pl.BlockSpec(memory_space=pltpu.VMEM),
        scratch_shapes=[
            pltpu.VMEM((1, n), jnp.float32),
            pltpu.VMEM((N_DEV, 1, n), jnp.float32),
            pltpu.SemaphoreType.DMA((N_DEV - 1,)),
            pltpu.SemaphoreType.DMA((N_DEV,)),
        ],
        compiler_params=pltpu.CompilerParams(collective_id=0),
    )(x)


# device time: 6400 ns/iter; 2.0581x vs baseline; 2.0581x over previous
import jax
import jax.numpy as jnp
from jax import lax
from jax.experimental import pallas as pl
from jax.experimental.pallas import tpu as pltpu

N_DEV = 32


def kernel(x):
    m_per, n = x.shape

    def body(x_ref, out_ref, own_ref, comm_ref, send_sems, recv_sems):
        my = lax.axis_index("i")
        barrier_sem = pltpu.get_barrier_semaphore()

        for d in range(1, N_DEV):

            @pl.when(my - d >= 0)
            def _(d=d):
                pl.semaphore_signal(
                    barrier_sem,
                    inc=1,
                    device_id=(my - d,),
                    device_id_type=pl.DeviceIdType.MESH,
                )

        own_ref[0, :] = jnp.sum(x_ref[...], axis=0)

        chunk = 128
        n_chunks = m_per // chunk
        split = 6
        row = lax.broadcasted_iota(jnp.int32, (chunk, chunk), 0)
        col = lax.broadcasted_iota(jnp.int32, (chunk, chunk), 1)
        tri = (row >= col).astype(jnp.bfloat16)
        running = jnp.zeros((1, n), jnp.float32)

        def do_chunk(c, running):
            seg = x_ref[c * chunk : (c + 1) * chunk, :].astype(jnp.bfloat16)
            scan_c = jax.lax.dot_general(
                tri, seg, (((1,), (0,)), ((), ())),
                preferred_element_type=jnp.float32,
            )
            out_ref[c * chunk : (c + 1) * chunk, :] = scan_c + running
            return running + scan_c[-1:, :]

        for c in range(split):
            running = do_chunk(c, running)

        for d in range(1, N_DEV):

            @pl.when(my + d < N_DEV)
            def _(d=d):
                pl.semaphore_wait(barrier_sem, 1)

        for d in range(1, N_DEV):

            @pl.when(my + d < N_DEV)
            def _(d=d):
                rdma = pltpu.make_async_remote_copy(
                    src_ref=own_ref,
                    dst_ref=comm_ref.at[my],
                    send_sem=send_sems.at[d - 1],
                    recv_sem=recv_sems.at[my],
                    device_id=(my + d,),
                    device_id_type=pl.DeviceIdType.MESH,
                )
                rdma.start()

        for c in range(split, n_chunks):
            running = do_chunk(c, running)

        for j in range(N_DEV - 1):

            @pl.when(j < my)
            def _(j=j):
                recv = pltpu.make_async_remote_copy(
                    src_ref=own_ref,
                    dst_ref=comm_ref.at[j],
                    send_sem=send_sems.at[0],
                    recv_sem=recv_sems.at[j],
                    device_id=(0,),
                    device_id_type=pl.DeviceIdType.MESH,
                )
                recv.wait_recv()

        slot_row = lax.broadcasted_iota(jnp.int32, (N_DEV, n), 0)
        totals = jnp.where(slot_row < my, comm_ref[:, 0, :], 0.0)
        prefix = jnp.sum(totals, axis=0, keepdims=True)

        out_ref[...] = out_ref[...] + prefix

        for d in range(1, N_DEV):

            @pl.when(my + d < N_DEV)
            def _(d=d):
                send = pltpu.make_async_remote_copy(
                    src_ref=own_ref,
                    dst_ref=comm_ref.at[my],
                    send_sem=send_sems.at[d - 1],
                    recv_sem=recv_sems.at[0],
                    device_id=(my + d,),
                    device_id_type=pl.DeviceIdType.MESH,
                )
                send.wait_send()

    return pl.pallas_call(
        body,
        out_shape=jax.ShapeDtypeStruct((m_per, n), jnp.float32),
        in_specs=[pl.BlockSpec(memory_space=pltpu.VMEM)],
        out_specs=pl.BlockSpec(memory_space=pltpu.VMEM),
        scratch_shapes=[
            pltpu.VMEM((1, n), jnp.float32),
            pltpu.VMEM((N_DEV, 1, n), jnp.float32),
            pltpu.SemaphoreType.DMA((N_DEV - 1,)),
            pltpu.SemaphoreType.DMA((N_DEV,)),
        ],
        compiler_params=pltpu.CompilerParams(collective_id=0),
    )(x)
